# Initial kernel scaffold; baseline (speedup 1.0000x reference)
import jax
import jax.numpy as jnp
from jax.experimental import pallas as pl


def kernel(x, edge_index, edge_attr, W, b):
    raise NotImplementedError("write your pallas kernel here")


# speedup vs baseline: 4.1773x; 4.1773x over previous
import functools

import jax
import jax.numpy as jnp
from jax import lax
from jax.experimental import pallas as pl
from jax.experimental.pallas import tpu as pltpu
from jax.experimental.pallas import tpu_sc as plsc

N_NODES = 10000
DIM = 128
NC = 2
NS = 16
NW = NC * NS
CHUNK = 128
ROWS_PER_TILE = N_NODES // NS

_mesh = plsc.VectorSubcoreMesh(core_axis_name="c", subcore_axis_name="s")


def _make_sc_aggregate(e_pad: int):
    per_w = e_pad // NW
    n_chunks = per_w // CHUNK

    @functools.partial(
        pl.kernel,
        out_type=jax.ShapeDtypeStruct((NC, N_NODES, DIM), jnp.float32),
        mesh=_mesh,
        scratch_types=[
            pltpu.VMEM((CHUNK,), jnp.int32),
            pltpu.VMEM((CHUNK,), jnp.int32),
            pltpu.VMEM((CHUNK,), jnp.float32),
            pltpu.VMEM((CHUNK, DIM), jnp.float32),
            pltpu.VMEM_SHARED((N_NODES, DIM), jnp.float32),
            pltpu.SemaphoreType.DMA,
        ],
    )
    def _sc_aggregate(x_hbm, src_hbm, dst_hbm, attr_hbm, zeros_hbm, part_hbm,
                      idx_s, idx_d, attr_v, rows_v, acc, sem):
        cid = lax.axis_index("c")
        sid = lax.axis_index("s")

        @pl.when(sid == 0)
        def _zero():
            pltpu.sync_copy(zeros_hbm, acc)

        plsc.subcore_barrier()

        base = (cid * NS + sid) * per_w

        def chunk_body(ci, carry):
            off = base + ci * CHUNK
            pltpu.sync_copy(src_hbm.at[pl.ds(off, CHUNK)], idx_s)
            pltpu.sync_copy(dst_hbm.at[pl.ds(off, CHUNK)], idx_d)
            pltpu.sync_copy(attr_hbm.at[pl.ds(off, CHUNK)], attr_v)
            pltpu.async_copy(x_hbm.at[idx_s], rows_v, sem).wait()

            def group_body(g, c2):
                a16 = attr_v[pl.ds(g * 16, 16)]
                for l in range(16):
                    av = jnp.full((16,), a16[l], dtype=jnp.float32)
                    e = g * 16 + l
                    for j in range(DIM // 16):
                        sl = pl.ds(j * 16, 16)
                        rows_v[e, sl] = rows_v[e, sl] * av
                return c2

            lax.fori_loop(0, CHUNK // 16, group_body, 0)
            pltpu.sync_copy(rows_v, acc.at[idx_d], add=True)
            return carry

        lax.fori_loop(0, n_chunks, chunk_body, 0)

        plsc.subcore_barrier()
        r0 = sid * 624

        @pl.when(sid < NS - 1)
        def _copy_main():
            pltpu.sync_copy(acc.at[pl.ds(r0, 624)],
                            part_hbm.at[cid, pl.ds(r0, 624)])

        @pl.when(sid == NS - 1)
        def _copy_last():
            pltpu.sync_copy(acc.at[pl.ds((NS - 1) * 624, 640)],
                            part_hbm.at[cid, pl.ds((NS - 1) * 624, 640)])

    return _sc_aggregate


BLK = 1000


def _tc_body(part_ref, w_ref, b_ref, o_ref):
    p = part_ref[0] + part_ref[1]
    y = lax.dot_general(p, w_ref[...], (((1,), (1,)), ((), ())),
                        preferred_element_type=jnp.float32)
    y = y + b_ref[...]
    o_ref[...] = jnp.where(y >= 0.0, y, 0.01 * y)


_tc_linear = pl.pallas_call(
    _tc_body,
    grid=(N_NODES // BLK,),
    in_specs=[
        pl.BlockSpec((NC, BLK, DIM), lambda i: (0, i, 0)),
        pl.BlockSpec((DIM, DIM), lambda i: (0, 0)),
        pl.BlockSpec((1, DIM), lambda i: (0, 0)),
    ],
    out_specs=pl.BlockSpec((BLK, DIM), lambda i: (i, 0)),
    out_shape=jax.ShapeDtypeStruct((N_NODES, DIM), jnp.float32),
)


def kernel(x, edge_index, edge_attr, W, b):
    src = edge_index[0].astype(jnp.int32)
    dst = edge_index[1].astype(jnp.int32)
    attr = edge_attr.astype(jnp.float32)
    n_e = src.shape[0]
    e_pad = -(-n_e // (NW * CHUNK)) * (NW * CHUNK)
    pad = e_pad - n_e
    if pad:
        src = jnp.pad(src, (0, pad))
        dst = jnp.pad(dst, (0, pad))
        attr = jnp.pad(attr, (0, pad))
    zeros = jnp.zeros((N_NODES, DIM), jnp.float32)
    part = _make_sc_aggregate(e_pad)(x, src, dst, attr, zeros)
    return _tc_linear(part, W, b.reshape(1, DIM))
